# baseline (speedup 1.0000x reference)
import functools

import jax
import jax.numpy as jnp
from jax import lax
from jax.experimental import pallas as pl
from jax.experimental.pallas import tpu as pltpu
from jax.experimental.pallas import tpu_sc as plsc

N = 10000
E = 320000
H = 128
K = 30
NP = 10240
NC = 2
NS = 16
NW = NC * NS
CHUNK = 128
NCH = 80
EPW = NCH * CHUNK
EP = NW * EPW
RING = 2
CH_SLOW = 44
CH_FAST = 116
NCHD = (NC * EP) // (NC * NS) // CHUNK
WS = 16
RPS = NP // NS
CHZ = 80
BR = 1024
NEG = -1e30

def _mesh():
    return plsc.VectorSubcoreMesh(core_axis_name="c", subcore_axis_name="s",
                                  num_cores=NC, num_subcores=NS)


def _wid():
    return lax.axis_index("s") * NC + lax.axis_index("c")


def _dot_bf16(a, b):
    return jnp.dot(a.astype(jnp.bfloat16), b.astype(jnp.bfloat16),
                   preferred_element_type=jnp.float32)


def _sc_prep_body(idx4, zp, ztab, zeros_w, ones_w,
                  x_out, deg2_out,
                  ix_v, zidx, zrows, ones_v, deg_sh, zsem, *sems):
    isem = sems
    cid = lax.axis_index("c")
    sid = lax.axis_index("s")
    wid = _wid()
    r0 = sid * RPS
    pltpu.sync_copy(zeros_w.at[pl.ds(r0, RPS)], deg_sh.at[pl.ds(r0, RPS)])
    pltpu.sync_copy(ones_w, ones_v)

    def fire_idx(j, ch):
        pltpu.async_copy(idx4.at[cid, sid, ch], ix_v.at[j], isem[j])

    def wait_idx(j):
        pltpu.make_async_copy(idx4.at[cid, sid, 0], ix_v.at[j],
                              isem[j]).wait()

    def scatter(j):
        pltpu.sync_copy(ones_v, deg_sh.at[ix_v.at[j]], add=True)

    for j in range(RING):
        fire_idx(j, j)
    plsc.subcore_barrier()

    def body(i, _):
        for j in range(RING):
            wait_idx(j)
            scatter(j)
            fire_idx(j, (i + 1) * RING + j)
        return ()

    lax.fori_loop(0, NCHD // RING - 1, body, (), unroll=False)
    for j in range(RING):
        wait_idx(j)
        scatter(j)

    def zbody(kz, _):
        zbase = pl.multiple_of(wid * (NP // NW) + kz * CHZ, CHZ)
        pltpu.sync_copy(zp.at[pl.ds(zbase, CHZ)], zidx)
        pltpu.async_copy(ztab.at[zidx], zrows, zsem).wait()
        pltpu.sync_copy(zrows, x_out.at[pl.ds(zbase, CHZ)])
        return ()

    lax.fori_loop(0, (NP // NW) // CHZ, zbody, (), unroll=False)

    plsc.subcore_barrier()
    pltpu.sync_copy(deg_sh.at[pl.ds(r0, RPS)], deg2_out.at[cid, pl.ds(r0, RPS)])


def _sc_prep(srcp, dstp, zp, ztab):
    idx4 = jnp.concatenate([srcp, dstp]).reshape(NC, NS, NCHD, CHUNK)
    zeros_w = jnp.zeros((NP, H), jnp.float32)
    ones_w = jnp.ones((CHUNK, H), jnp.float32)
    f = pl.kernel(
        _sc_prep_body,
        out_type=(jax.ShapeDtypeStruct((NP, H), jnp.float32),
                  jax.ShapeDtypeStruct((NC, NP, H), jnp.float32)),
        mesh=_mesh(),
        scratch_types=[
            pltpu.VMEM((RING, CHUNK), jnp.int32),
            pltpu.VMEM((CHZ,), jnp.int32),
            pltpu.VMEM((CHZ, H), jnp.float32),
            pltpu.VMEM((CHUNK, H), jnp.float32),
            pltpu.VMEM_SHARED((NP, H), jnp.float32),
            pltpu.SemaphoreType.DMA,
        ] + [pltpu.SemaphoreType.DMA] * RING,
    )
    return f(idx4, zp, ztab, zeros_w, ones_w)


def _sc_segsum_body(h, src3, dst3, zeros_w, agg_out,
                    gidx, didx, rows_v, agg_sh, *sems):
    isem = sems[:RING]
    gsem = sems[RING:]
    cid = lax.axis_index("c")
    sid = lax.axis_index("s")
    r0 = sid * RPS
    off = jnp.where(cid == 0, 0, CH_SLOW)
    cnt = jnp.where(cid == 0, CH_SLOW, CH_FAST)
    pltpu.sync_copy(zeros_w.at[pl.ds(r0, RPS)], agg_sh.at[pl.ds(r0, RPS)])

    def buf(j):
        return rows_v.at[pl.ds(j * CHUNK, CHUNK)]

    def fire_idx(j, ch):
        pltpu.async_copy(src3.at[sid, off + ch], gidx.at[j], isem[j])
        pltpu.async_copy(dst3.at[sid, off + ch], didx.at[j], isem[j])

    def wait_idx(j):
        pltpu.make_async_copy(src3.at[sid, 0], gidx.at[j], isem[j]).wait()
        pltpu.make_async_copy(dst3.at[sid, 0], didx.at[j], isem[j]).wait()

    def fire_gather(j):
        pltpu.async_copy(h.at[gidx.at[j]], buf(j), gsem[j])

    def wait_gather(j):
        pltpu.make_async_copy(h.at[gidx.at[j]], buf(j), gsem[j]).wait()

    def scatter(j):
        pltpu.sync_copy(buf(j), agg_sh.at[didx.at[j]], add=True)

    for j in range(RING):
        fire_idx(j, j)
    plsc.subcore_barrier()
    for j in range(RING):
        wait_idx(j)
        fire_gather(j)

    def body(i, _):
        for j in range(RING):
            wait_gather(j)
            scatter(j)
        for j in range(RING):
            fire_idx(j, (i + 1) * RING + j)
        for j in range(RING):
            wait_idx(j)
            fire_gather(j)
        return ()

    lax.fori_loop(0, cnt // RING - 1, body, (), unroll=False)
    for j in range(RING):
        wait_gather(j)
        scatter(j)
    plsc.subcore_barrier()
    pltpu.sync_copy(agg_sh.at[pl.ds(r0, RPS)], agg_out.at[cid, pl.ds(r0, RPS)])


def _sc_segsum(h, srcp, dstp, width):
    src3 = srcp.reshape(NS, NC * NCH, CHUNK)
    dst3 = dstp.reshape(NS, NC * NCH, CHUNK)
    zeros_w = jnp.zeros((NP, width), jnp.float32)
    f = pl.kernel(
        _sc_segsum_body,
        out_type=jax.ShapeDtypeStruct((NC, NP, width), jnp.float32),
        mesh=_mesh(),
        scratch_types=[
            pltpu.VMEM((RING, CHUNK), jnp.int32),
            pltpu.VMEM((RING, CHUNK), jnp.int32),
            pltpu.VMEM((RING * CHUNK, width), jnp.float32),
            pltpu.VMEM_SHARED((NP, width), jnp.float32),
        ] + [pltpu.SemaphoreType.DMA] * (2 * RING),
    )
    return f(h, src3, dst3, zeros_w)


def _tc_first_body(x_ref, deg_ref, w_ref, h_ref, dov_ref, div_ref):
    dov = lax.rsqrt(jnp.maximum(deg_ref[0, :, 0], 1.0))
    div = lax.rsqrt(jnp.maximum(deg_ref[1, :, 0], 1.0))
    dov_ref[...] = dov
    div_ref[...] = div
    h_ref[...] = _dot_bf16(x_ref[...] * dov[:, None], w_ref[...])


def _tc_first(x, deg2, W0):
    return pl.pallas_call(
        _tc_first_body,
        grid=(NP // BR,),
        in_specs=[
            pl.BlockSpec((BR, H), lambda i: (i, 0)),
            pl.BlockSpec((NC, BR, H), lambda i: (0, i, 0)),
            pl.BlockSpec((H, H), lambda i: (0, 0)),
        ],
        out_specs=[
            pl.BlockSpec((BR, H), lambda i: (i, 0)),
            pl.BlockSpec((BR,), lambda i: (i,)),
            pl.BlockSpec((BR,), lambda i: (i,)),
        ],
        out_shape=[
            jax.ShapeDtypeStruct((NP, H), jnp.float32),
            jax.ShapeDtypeStruct((NP,), jnp.float32),
            jax.ShapeDtypeStruct((NP,), jnp.float32),
        ],
    )(x, deg2, W0)


def _tc_mid_body(agg_ref, di_ref, do_ref, b_ref, w_ref, x_ref, h_ref):
    agg = agg_ref[0] + agg_ref[1]
    xv = jnp.tanh(agg * di_ref[...][:, None] + b_ref[0, :][None, :])
    x_ref[...] = xv
    h_ref[...] = _dot_bf16(xv * do_ref[...][:, None], w_ref[...])


def _tc_mid(aggp, dinv_in, dinv_out, b, W):
    return pl.pallas_call(
        _tc_mid_body,
        grid=(NP // BR,),
        in_specs=[
            pl.BlockSpec((NC, BR, H), lambda i: (0, i, 0)),
            pl.BlockSpec((BR,), lambda i: (i,)),
            pl.BlockSpec((BR,), lambda i: (i,)),
            pl.BlockSpec((1, H), lambda i: (0, 0)),
            pl.BlockSpec((H, H), lambda i: (0, 0)),
        ],
        out_specs=[
            pl.BlockSpec((BR, H), lambda i: (i, 0)),
            pl.BlockSpec((BR, H), lambda i: (i, 0)),
        ],
        out_shape=[
            jax.ShapeDtypeStruct((NP, H), jnp.float32),
            jax.ShapeDtypeStruct((NP, H), jnp.float32),
        ],
    )(aggp, dinv_in, dinv_out, b, W)


def _tc_last_body(agg_ref, di_ref, do_ref, b_ref, x_ref, h_ref):
    agg = agg_ref[0] + agg_ref[1]
    xv = jnp.tanh(agg * di_ref[...][:, None] + b_ref[0, :][None, :])
    x_ref[...] = xv
    h_ref[...] = (xv * do_ref[...][:, None]).astype(jnp.bfloat16).astype(
        jnp.float32)


def _tc_last(aggp, dinv_in, dinv_out, b):
    return pl.pallas_call(
        _tc_last_body,
        grid=(NP // BR,),
        in_specs=[
            pl.BlockSpec((NC, BR, H), lambda i: (0, i, 0)),
            pl.BlockSpec((BR,), lambda i: (i,)),
            pl.BlockSpec((BR,), lambda i: (i,)),
            pl.BlockSpec((1, H), lambda i: (0, 0)),
        ],
        out_specs=[
            pl.BlockSpec((BR, H), lambda i: (i, 0)),
            pl.BlockSpec((BR, H), lambda i: (i, 0)),
        ],
        out_shape=[
            jax.ShapeDtypeStruct((NP, H), jnp.float32),
            jax.ShapeDtypeStruct((NP, H), jnp.float32),
        ],
    )(aggp, dinv_in, dinv_out, b)


def _bitonic_sort_rows(x):
    rows, width = x.shape
    lane = lax.broadcasted_iota(jnp.int32, (rows, width), 1)
    size = 2
    while size <= width:
        stride = size // 2
        while stride >= 1:
            up = (lane & size) == 0
            lo = (lane & stride) == 0
            partner = jnp.where(lo,
                                jnp.roll(x, -stride, axis=1),
                                jnp.roll(x, stride, axis=1))
            mn = jnp.minimum(x, partner)
            mx = jnp.maximum(x, partner)
            x = jnp.where(lo == up, mn, mx)
            stride //= 2
        size *= 2
    return x


def _tc_head_body(a3_ref, di_ref, x1_ref, x2_ref, x3_ref, b3_ref, w3_ref,
                  w1hT_ref, b1h_ref, w2rT_ref, b2h_ref, lw_ref, lb1_ref,
                  l2T_ref, lb2_ref, out_ref, p_scr):
    w3bf = w3_ref[0, :].astype(jnp.bfloat16).astype(jnp.float32)
    agg3v = jnp.sum((a3_ref[0] + a3_ref[1]) * w3bf[None, :], axis=1)
    x4 = jnp.tanh(agg3v * di_ref[...] + b3_ref[0, 0])
    keys = jnp.maximum(jnp.max(x1_ref[...], axis=1),
                       jnp.max(x2_ref[...], axis=1))
    keys = jnp.maximum(keys, jnp.max(x3_ref[...], axis=1))
    keys = jnp.maximum(keys, x4)
    iota = lax.broadcasted_iota(jnp.int32, (NP,), 0)
    keys = jnp.where(iota < N, keys, NEG)

    p_scr[...] = jnp.zeros((32, NP), jnp.float32)
    kcur = keys
    for k in range(K):
        m = jnp.max(kcur)
        idx = jnp.min(jnp.where(kcur == m, iota, NP))
        oh = iota == idx
        p_scr[pl.ds(k, 1), :] = oh.astype(jnp.float32)[None, :]
        kcur = jnp.where(oh, NEG, kcur)

    P = p_scr[...]
    sel1 = jnp.dot(P, x1_ref[...], preferred_element_type=jnp.float32)
    sel2 = jnp.dot(P, x2_ref[...], preferred_element_type=jnp.float32)
    sel3 = jnp.dot(P, x3_ref[...], preferred_element_type=jnp.float32)
    x4b = jnp.broadcast_to(x4[:, None], (NP, 8))
    sel4 = jnp.dot(P, x4b, preferred_element_type=jnp.float32)
    feat = jnp.concatenate([sel1, sel2, sel3, sel4[:, 0:1]], axis=1)
    featp = jnp.concatenate(
        [feat, jnp.full((32, 512 - 3 * H - 1), 3e38, jnp.float32)], axis=1)
    srt = _bitonic_sort_rows(featp)
    srt = srt[0:K, 0:3 * H + 1]

    t1 = jnp.maximum(_dot_bf16(srt, w1hT_ref[...])
                     + b1h_ref[0, :][None, :], 0.0)
    tp = jnp.max(t1.reshape(K // 2, 2, 16), axis=1)
    M = jnp.concatenate([tp[j:j + 11, :] for j in range(5)], axis=1)
    t2 = jnp.maximum(_dot_bf16(M, w2rT_ref[...])
                     + b2h_ref[0, :][None, :], 0.0)
    acc = lb1_ref[...]
    for j in range(11):
        acc = acc + _dot_bf16(t2[j:j + 1, :], lw_ref[j])
    o1 = jnp.maximum(acc, 0.0)
    out_ref[...] = _dot_bf16(o1, l2T_ref[...]) + lb2_ref[...]


def _tc_head(agg3p, dinv_in, x1, x2, x3, b3s, w3row, w1hT, b1h, w2rT, b2h,
             lw, lb1, l2T, lb2):
    return pl.pallas_call(
        _tc_head_body,
        out_shape=jax.ShapeDtypeStruct((1, 1), jnp.float32),
        scratch_shapes=[pltpu.VMEM((32, NP), jnp.float32)],
    )(agg3p, dinv_in, x1, x2, x3, b3s, w3row, w1hT, b1h, w2rT, b2h, lw, lb1,
      l2T, lb2)


def kernel(edge_index, z, z_table, W0, b0, W1, b1, W2, b2, W3, b3,
           conv1_w, conv1_b, conv2_w, conv2_b, lin1_w, lin1_b, lin2_w,
           lin2_b):
    src = edge_index[0]
    dst = edge_index[1]
    pad = jnp.full((EP - E,), N, jnp.int32)
    srcp = jnp.concatenate([src, pad])
    dstp = jnp.concatenate([dst, pad])
    zp = jnp.concatenate([z, jnp.zeros((NP - N,), jnp.int32)])

    x0, deg2 = _sc_prep(srcp, dstp, zp, z_table)
    h0, dinv_out, dinv_in = _tc_first(x0, deg2, W0)

    agg0 = _sc_segsum(h0, srcp, dstp, H)
    x1, h1 = _tc_mid(agg0, dinv_in, dinv_out, b0.reshape(1, H), W1)
    agg1 = _sc_segsum(h1, srcp, dstp, H)
    x2, h2 = _tc_mid(agg1, dinv_in, dinv_out, b1.reshape(1, H), W2)
    agg2 = _sc_segsum(h2, srcp, dstp, H)
    x3, x3n = _tc_last(agg2, dinv_in, dinv_out, b2.reshape(1, H))
    agg3 = _sc_segsum(x3n, srcp, dstp, H)

    w1hT = conv1_w[:, 0, :].T
    b1h = conv1_b.reshape(1, 16)
    w2rT = conv2_w.transpose(0, 2, 1).reshape(32, 80).T
    b2h = conv2_b.reshape(1, 32)
    lw = lin1_w.reshape(128, 32, 11).transpose(2, 1, 0)
    lb1 = lin1_b.reshape(1, 128)
    l2T = lin2_w.T
    lb2 = lin2_b.reshape(1, 1)
    return _tc_head(agg3, dinv_in, x1, x2, x3, b3.reshape(1, 1),
                    W3[:, 0].reshape(1, H), w1hT, b1h, w2rT, b2h, lw, lb1,
                    l2T, lb2)


# speedup vs baseline: 1.2360x; 1.2360x over previous
import functools

import jax
import jax.numpy as jnp
from jax import lax
from jax.experimental import pallas as pl
from jax.experimental.pallas import tpu as pltpu
from jax.experimental.pallas import tpu_sc as plsc

N = 10000
E = 320000
H = 128
K = 30
NP = 10240
NC = 2
NS = 16
NW = NC * NS
CHUNK = 128
NCH = 80
EPW = NCH * CHUNK
EP = NW * EPW
RING = 2
CH_SLOW = 116
CH_FAST = 44
NCHD = (NC * EP) // (NC * NS) // CHUNK
WS = 16
RPS = NP // NS
CHZ = 80
BR = 1024
NEG = -1e30

def _mesh():
    return plsc.VectorSubcoreMesh(core_axis_name="c", subcore_axis_name="s",
                                  num_cores=NC, num_subcores=NS)


def _wid():
    return lax.axis_index("s") * NC + lax.axis_index("c")


def _dot_bf16(a, b):
    return jnp.dot(a.astype(jnp.bfloat16), b.astype(jnp.bfloat16),
                   preferred_element_type=jnp.float32)


def _sc_prep_body(idx4, zp, ztab, zeros_w, ones_w,
                  x_out, deg2_out,
                  ix_v, zidx, zrows, ones_v, deg_sh, zsem, *sems):
    isem = sems
    cid = lax.axis_index("c")
    sid = lax.axis_index("s")
    wid = _wid()
    r0 = sid * RPS
    pltpu.sync_copy(zeros_w.at[pl.ds(r0, RPS)], deg_sh.at[pl.ds(r0, RPS)])
    pltpu.sync_copy(ones_w, ones_v)

    def fire_idx(j, ch):
        pltpu.async_copy(idx4.at[cid, sid, ch], ix_v.at[j], isem[j])

    def wait_idx(j):
        pltpu.make_async_copy(idx4.at[cid, sid, 0], ix_v.at[j],
                              isem[j]).wait()

    def scatter(j):
        pltpu.sync_copy(ones_v, deg_sh.at[ix_v.at[j]], add=True)

    for j in range(RING):
        fire_idx(j, j)
    plsc.subcore_barrier()

    def body(i, _):
        for j in range(RING):
            wait_idx(j)
            scatter(j)
            fire_idx(j, (i + 1) * RING + j)
        return ()

    lax.fori_loop(0, NCHD // RING - 1, body, (), unroll=False)
    for j in range(RING):
        wait_idx(j)
        scatter(j)

    def zbody(kz, _):
        zbase = pl.multiple_of(wid * (NP // NW) + kz * CHZ, CHZ)
        pltpu.sync_copy(zp.at[pl.ds(zbase, CHZ)], zidx)
        pltpu.async_copy(ztab.at[zidx], zrows, zsem).wait()
        pltpu.sync_copy(zrows, x_out.at[pl.ds(zbase, CHZ)])
        return ()

    lax.fori_loop(0, (NP // NW) // CHZ, zbody, (), unroll=False)

    plsc.subcore_barrier()
    pltpu.sync_copy(deg_sh.at[pl.ds(r0, RPS)], deg2_out.at[cid, pl.ds(r0, RPS)])


def _sc_prep(srcp, dstp, zp, ztab):
    idx4 = jnp.concatenate([srcp, dstp]).reshape(NC, NS, NCHD, CHUNK)
    zeros_w = jnp.zeros((NP, H), jnp.float32)
    ones_w = jnp.ones((CHUNK, H), jnp.float32)
    f = pl.kernel(
        _sc_prep_body,
        out_type=(jax.ShapeDtypeStruct((NP, H), jnp.float32),
                  jax.ShapeDtypeStruct((NC, NP, H), jnp.float32)),
        mesh=_mesh(),
        scratch_types=[
            pltpu.VMEM((RING, CHUNK), jnp.int32),
            pltpu.VMEM((CHZ,), jnp.int32),
            pltpu.VMEM((CHZ, H), jnp.float32),
            pltpu.VMEM((CHUNK, H), jnp.float32),
            pltpu.VMEM_SHARED((NP, H), jnp.float32),
            pltpu.SemaphoreType.DMA,
        ] + [pltpu.SemaphoreType.DMA] * RING,
    )
    return f(idx4, zp, ztab, zeros_w, ones_w)


def _sc_segsum_body(h, src3, dst3, zeros_w, agg_out,
                    gidx, didx, rows_v, agg_sh, *sems):
    isem = sems[:RING]
    gsem = sems[RING:]
    cid = lax.axis_index("c")
    sid = lax.axis_index("s")
    r0 = sid * RPS
    off = jnp.where(cid == 0, 0, CH_SLOW)
    cnt = jnp.where(cid == 0, CH_SLOW, CH_FAST)
    pltpu.sync_copy(zeros_w.at[pl.ds(r0, RPS)], agg_sh.at[pl.ds(r0, RPS)])

    def buf(j):
        return rows_v.at[pl.ds(j * CHUNK, CHUNK)]

    def fire_idx(j, ch):
        pltpu.async_copy(src3.at[sid, off + ch], gidx.at[j], isem[j])
        pltpu.async_copy(dst3.at[sid, off + ch], didx.at[j], isem[j])

    def wait_idx(j):
        pltpu.make_async_copy(src3.at[sid, 0], gidx.at[j], isem[j]).wait()
        pltpu.make_async_copy(dst3.at[sid, 0], didx.at[j], isem[j]).wait()

    def fire_gather(j):
        pltpu.async_copy(h.at[gidx.at[j]], buf(j), gsem[j])

    def wait_gather(j):
        pltpu.make_async_copy(h.at[gidx.at[j]], buf(j), gsem[j]).wait()

    def scatter(j):
        pltpu.sync_copy(buf(j), agg_sh.at[didx.at[j]], add=True)

    for j in range(RING):
        fire_idx(j, j)
    plsc.subcore_barrier()
    for j in range(RING):
        wait_idx(j)
        fire_gather(j)

    def body(i, _):
        for j in range(RING):
            wait_gather(j)
            scatter(j)
        for j in range(RING):
            fire_idx(j, (i + 1) * RING + j)
        for j in range(RING):
            wait_idx(j)
            fire_gather(j)
        return ()

    lax.fori_loop(0, cnt // RING - 1, body, (), unroll=False)
    for j in range(RING):
        wait_gather(j)
        scatter(j)
    plsc.subcore_barrier()
    pltpu.sync_copy(agg_sh.at[pl.ds(r0, RPS)], agg_out.at[cid, pl.ds(r0, RPS)])


def _sc_segsum(h, srcp, dstp, width):
    src3 = srcp.reshape(NS, NC * NCH, CHUNK)
    dst3 = dstp.reshape(NS, NC * NCH, CHUNK)
    zeros_w = jnp.zeros((NP, width), jnp.float32)
    f = pl.kernel(
        _sc_segsum_body,
        out_type=jax.ShapeDtypeStruct((NC, NP, width), jnp.float32),
        mesh=_mesh(),
        scratch_types=[
            pltpu.VMEM((RING, CHUNK), jnp.int32),
            pltpu.VMEM((RING, CHUNK), jnp.int32),
            pltpu.VMEM((RING * CHUNK, width), jnp.float32),
            pltpu.VMEM_SHARED((NP, width), jnp.float32),
        ] + [pltpu.SemaphoreType.DMA] * (2 * RING),
    )
    return f(h, src3, dst3, zeros_w)


def _tc_first_body(x_ref, deg_ref, w_ref, h_ref, dov_ref, div_ref):
    dov = lax.rsqrt(jnp.maximum(deg_ref[0, :, 0], 1.0))
    div = lax.rsqrt(jnp.maximum(deg_ref[1, :, 0], 1.0))
    dov_ref[...] = dov
    div_ref[...] = div
    h_ref[...] = _dot_bf16(x_ref[...] * dov[:, None], w_ref[...])


def _tc_first(x, deg2, W0):
    return pl.pallas_call(
        _tc_first_body,
        grid=(NP // BR,),
        in_specs=[
            pl.BlockSpec((BR, H), lambda i: (i, 0)),
            pl.BlockSpec((NC, BR, H), lambda i: (0, i, 0)),
            pl.BlockSpec((H, H), lambda i: (0, 0)),
        ],
        out_specs=[
            pl.BlockSpec((BR, H), lambda i: (i, 0)),
            pl.BlockSpec((BR,), lambda i: (i,)),
            pl.BlockSpec((BR,), lambda i: (i,)),
        ],
        out_shape=[
            jax.ShapeDtypeStruct((NP, H), jnp.float32),
            jax.ShapeDtypeStruct((NP,), jnp.float32),
            jax.ShapeDtypeStruct((NP,), jnp.float32),
        ],
    )(x, deg2, W0)


def _tc_mid_body(agg_ref, di_ref, do_ref, b_ref, w_ref, x_ref, h_ref):
    agg = agg_ref[0] + agg_ref[1]
    xv = jnp.tanh(agg * di_ref[...][:, None] + b_ref[0, :][None, :])
    x_ref[...] = xv
    h_ref[...] = _dot_bf16(xv * do_ref[...][:, None], w_ref[...])


def _tc_mid(aggp, dinv_in, dinv_out, b, W):
    return pl.pallas_call(
        _tc_mid_body,
        grid=(NP // BR,),
        in_specs=[
            pl.BlockSpec((NC, BR, H), lambda i: (0, i, 0)),
            pl.BlockSpec((BR,), lambda i: (i,)),
            pl.BlockSpec((BR,), lambda i: (i,)),
            pl.BlockSpec((1, H), lambda i: (0, 0)),
            pl.BlockSpec((H, H), lambda i: (0, 0)),
        ],
        out_specs=[
            pl.BlockSpec((BR, H), lambda i: (i, 0)),
            pl.BlockSpec((BR, H), lambda i: (i, 0)),
        ],
        out_shape=[
            jax.ShapeDtypeStruct((NP, H), jnp.float32),
            jax.ShapeDtypeStruct((NP, H), jnp.float32),
        ],
    )(aggp, dinv_in, dinv_out, b, W)


def _tc_last_body(agg_ref, di_ref, do_ref, b_ref, x_ref, h_ref):
    agg = agg_ref[0] + agg_ref[1]
    xv = jnp.tanh(agg * di_ref[...][:, None] + b_ref[0, :][None, :])
    x_ref[...] = xv
    h_ref[...] = (xv * do_ref[...][:, None]).astype(jnp.bfloat16).astype(
        jnp.float32)


def _tc_last(aggp, dinv_in, dinv_out, b):
    return pl.pallas_call(
        _tc_last_body,
        grid=(NP // BR,),
        in_specs=[
            pl.BlockSpec((NC, BR, H), lambda i: (0, i, 0)),
            pl.BlockSpec((BR,), lambda i: (i,)),
            pl.BlockSpec((BR,), lambda i: (i,)),
            pl.BlockSpec((1, H), lambda i: (0, 0)),
        ],
        out_specs=[
            pl.BlockSpec((BR, H), lambda i: (i, 0)),
            pl.BlockSpec((BR, H), lambda i: (i, 0)),
        ],
        out_shape=[
            jax.ShapeDtypeStruct((NP, H), jnp.float32),
            jax.ShapeDtypeStruct((NP, H), jnp.float32),
        ],
    )(aggp, dinv_in, dinv_out, b)


def _bitonic_sort_rows(x):
    rows, width = x.shape
    lane = lax.broadcasted_iota(jnp.int32, (rows, width), 1)
    size = 2
    while size <= width:
        stride = size // 2
        while stride >= 1:
            up = (lane & size) == 0
            lo = (lane & stride) == 0
            partner = jnp.where(lo,
                                jnp.roll(x, -stride, axis=1),
                                jnp.roll(x, stride, axis=1))
            mn = jnp.minimum(x, partner)
            mx = jnp.maximum(x, partner)
            x = jnp.where(lo == up, mn, mx)
            stride //= 2
        size *= 2
    return x


def _tc_head_body(a3_ref, di_ref, x1_ref, x2_ref, x3_ref, b3_ref, w3_ref,
                  w1hT_ref, b1h_ref, w2rT_ref, b2h_ref, lw_ref, lb1_ref,
                  l2T_ref, lb2_ref, out_ref, p_scr):
    w3bf = w3_ref[0, :].astype(jnp.bfloat16).astype(jnp.float32)
    agg3v = jnp.sum((a3_ref[0] + a3_ref[1]) * w3bf[None, :], axis=1)
    x4 = jnp.tanh(agg3v * di_ref[...] + b3_ref[0, 0])
    keys = jnp.maximum(jnp.max(x1_ref[...], axis=1),
                       jnp.max(x2_ref[...], axis=1))
    keys = jnp.maximum(keys, jnp.max(x3_ref[...], axis=1))
    keys = jnp.maximum(keys, x4)
    iota = lax.broadcasted_iota(jnp.int32, (NP,), 0)
    keys = jnp.where(iota < N, keys, NEG)

    p_scr[...] = jnp.zeros((32, NP), jnp.float32)
    kcur = keys
    for k in range(K):
        m = jnp.max(kcur)
        idx = jnp.min(jnp.where(kcur == m, iota, NP))
        oh = iota == idx
        p_scr[pl.ds(k, 1), :] = oh.astype(jnp.float32)[None, :]
        kcur = jnp.where(oh, NEG, kcur)

    P = p_scr[...]
    sel1 = jnp.dot(P, x1_ref[...], preferred_element_type=jnp.float32)
    sel2 = jnp.dot(P, x2_ref[...], preferred_element_type=jnp.float32)
    sel3 = jnp.dot(P, x3_ref[...], preferred_element_type=jnp.float32)
    x4b = jnp.broadcast_to(x4[:, None], (NP, 8))
    sel4 = jnp.dot(P, x4b, preferred_element_type=jnp.float32)
    feat = jnp.concatenate([sel1, sel2, sel3, sel4[:, 0:1]], axis=1)
    featp = jnp.concatenate(
        [feat, jnp.full((32, 512 - 3 * H - 1), 3e38, jnp.float32)], axis=1)
    srt = _bitonic_sort_rows(featp)
    srt = srt[0:K, 0:3 * H + 1]

    t1 = jnp.maximum(_dot_bf16(srt, w1hT_ref[...])
                     + b1h_ref[0, :][None, :], 0.0)
    tp = jnp.max(t1.reshape(K // 2, 2, 16), axis=1)
    M = jnp.concatenate([tp[j:j + 11, :] for j in range(5)], axis=1)
    t2 = jnp.maximum(_dot_bf16(M, w2rT_ref[...])
                     + b2h_ref[0, :][None, :], 0.0)
    acc = lb1_ref[...]
    for j in range(11):
        acc = acc + _dot_bf16(t2[j:j + 1, :], lw_ref[j])
    o1 = jnp.maximum(acc, 0.0)
    out_ref[...] = _dot_bf16(o1, l2T_ref[...]) + lb2_ref[...]


def _tc_head(agg3p, dinv_in, x1, x2, x3, b3s, w3row, w1hT, b1h, w2rT, b2h,
             lw, lb1, l2T, lb2):
    return pl.pallas_call(
        _tc_head_body,
        out_shape=jax.ShapeDtypeStruct((1, 1), jnp.float32),
        scratch_shapes=[pltpu.VMEM((32, NP), jnp.float32)],
    )(agg3p, dinv_in, x1, x2, x3, b3s, w3row, w1hT, b1h, w2rT, b2h, lw, lb1,
      l2T, lb2)


def kernel(edge_index, z, z_table, W0, b0, W1, b1, W2, b2, W3, b3,
           conv1_w, conv1_b, conv2_w, conv2_b, lin1_w, lin1_b, lin2_w,
           lin2_b):
    src = edge_index[0]
    dst = edge_index[1]
    pad = jnp.full((EP - E,), N, jnp.int32)
    srcp = jnp.concatenate([src, pad])
    dstp = jnp.concatenate([dst, pad])
    zp = jnp.concatenate([z, jnp.zeros((NP - N,), jnp.int32)])

    x0, deg2 = _sc_prep(srcp, dstp, zp, z_table)
    h0, dinv_out, dinv_in = _tc_first(x0, deg2, W0)

    agg0 = _sc_segsum(h0, srcp, dstp, H)
    x1, h1 = _tc_mid(agg0, dinv_in, dinv_out, b0.reshape(1, H), W1)
    agg1 = _sc_segsum(h1, srcp, dstp, H)
    x2, h2 = _tc_mid(agg1, dinv_in, dinv_out, b1.reshape(1, H), W2)
    agg2 = _sc_segsum(h2, srcp, dstp, H)
    x3, x3n = _tc_last(agg2, dinv_in, dinv_out, b2.reshape(1, H))
    agg3 = _sc_segsum(x3n, srcp, dstp, H)

    w1hT = conv1_w[:, 0, :].T
    b1h = conv1_b.reshape(1, 16)
    w2rT = conv2_w.transpose(0, 2, 1).reshape(32, 80).T
    b2h = conv2_b.reshape(1, 32)
    lw = lin1_w.reshape(128, 32, 11).transpose(2, 1, 0)
    lb1 = lin1_b.reshape(1, 128)
    l2T = lin2_w.T
    lb2 = lin2_b.reshape(1, 1)
    return _tc_head(agg3, dinv_in, x1, x2, x3, b3.reshape(1, 1),
                    W3[:, 0].reshape(1, H), w1hT, b1h, w2rT, b2h, lw, lb1,
                    l2T, lb2)
